# baseline (speedup 1.0000x reference)
import functools
import math

import jax
import jax.numpy as jnp
import numpy as np
from jax import lax
from jax.experimental import pallas as pl
from jax.experimental.pallas import tpu as pltpu
from jax.experimental.pallas import tpu_sc as plsc

_VOCAB = 30522
_EMBED = 768
_MAX_LEN = 512
_NSEG = 3

_NCORE = 2
_NSUB = 16
_NW = _NCORE * _NSUB
_CHUNK = 32
_LANES = 16
_NVEC = _EMBED // _LANES


def _make_pe_np(d_model, max_len):
    pe = np.zeros((max_len, d_model), dtype=np.float32)
    position = np.arange(0, max_len, dtype=np.float32)[:, None]
    div_term = np.exp(
        np.arange(0, d_model, 2, dtype=np.float32) * -(math.log(10000.0) / d_model)
    )
    pe[:, 0::2] = np.sin(position * div_term)
    pe[:, 1::2] = np.cos(position * div_term)
    return pe


_PE = _make_pe_np(_EMBED, _MAX_LEN)


def _sc_body(seq_hbm, segid_hbm, tok_hbm, segflat_hbm, pe_hbm, out_hbm,
             tokidx_v, segid_v, segflat_v,
             tokbuf0, tokbuf1, pebuf0, pebuf1, pe_sh,
             sem_t0, sem_t1, sem_p0, sem_p1, sem_o0, sem_o1,
             rows_per_w=None):
    cid = lax.axis_index("c")
    sid = lax.axis_index("s")
    wid = sid * _NCORE + cid
    s0 = cid * rows_per_w
    base = wid * rows_per_w
    nchunk = rows_per_w // _CHUNK

    tok = [tokbuf0, tokbuf1]
    peb = [pebuf0, pebuf1]
    sem_t = [sem_t0, sem_t1]
    sem_p = [sem_p0, sem_p1]
    sem_o = [sem_o0, sem_o1]

    pltpu.sync_copy(seq_hbm.at[sid, pl.ds(s0, rows_per_w)], tokidx_v)
    pltpu.sync_copy(segid_hbm.at[sid, pl.ds(s0, rows_per_w)], segid_v)

    def start_tok(c, slot):
        return pltpu.async_copy(
            tok_hbm.at[tokidx_v.at[pl.ds(c * _CHUNK, _CHUNK)]],
            tok[slot], sem_t[slot])

    def start_pe(c, slot):
        return pltpu.async_copy(
            pe_sh.at[pl.ds(s0 + c * _CHUNK, _CHUNK)], peb[slot], sem_p[slot])

    cp_t0 = start_tok(0, 0)

    pltpu.sync_copy(segflat_hbm, segflat_v)
    rows_per_sub = _MAX_LEN // _NSUB
    pltpu.sync_copy(pe_hbm.at[pl.ds(sid * rows_per_sub, rows_per_sub)],
                    pe_sh.at[pl.ds(sid * rows_per_sub, rows_per_sub)])
    plsc.subcore_barrier()

    lanes = lax.iota(jnp.int32, _LANES)
    zeros16 = jnp.zeros((_LANES,), jnp.int32)

    in_cp = {0: (cp_t0, start_pe(0, 0))}
    out_cp = [None, None]

    for c in range(nchunk):
        cur = c & 1
        nxt = cur ^ 1
        if c + 1 < nchunk:
            if out_cp[nxt] is not None:
                out_cp[nxt].wait()
                out_cp[nxt] = None
            in_cp[c + 1] = (start_tok(c + 1, nxt), start_pe(c + 1, nxt))
        cp_t, cp_p = in_cp.pop(c)
        cp_t.wait()
        cp_p.wait()

        tbuf = tok[cur]
        pbuf = peb[cur]

        def row_body(r, carry, tbuf=tbuf, pbuf=pbuf, c=c):
            gvec = plsc.load_gather(segid_v, [zeros16 + (c * _CHUNK + r)])
            segbase = gvec * _EMBED + lanes
            for j in range(_NVEC):
                sl = pl.ds(j * _LANES, _LANES)
                segv = plsc.load_gather(segflat_v, [segbase + j * _LANES])
                plsc.addupdate(tbuf.at[r, sl], pbuf[r, sl] + segv)
            return carry

        lax.fori_loop(0, _CHUNK, row_body, 0)
        out_cp[cur] = pltpu.async_copy(
            tbuf, out_hbm.at[pl.ds(base + c * _CHUNK, _CHUNK)], sem_o[cur])

    for cp in out_cp:
        if cp is not None:
            cp.wait()


def kernel(sequence, segment_label, token_table, segment_table):
    b, s = sequence.shape
    n = b * s
    rows_per_w = n // _NW
    pe = jnp.asarray(_PE)
    segflat = segment_table.reshape(-1)
    mesh = plsc.VectorSubcoreMesh(core_axis_name="c", subcore_axis_name="s")
    out = pl.kernel(
        functools.partial(_sc_body, rows_per_w=rows_per_w),
        out_type=jax.ShapeDtypeStruct((n, _EMBED), jnp.float32),
        mesh=mesh,
        compiler_params=pltpu.CompilerParams(needs_layout_passes=False),
        scratch_types=[
            pltpu.VMEM((rows_per_w,), jnp.int32),
            pltpu.VMEM((rows_per_w,), jnp.int32),
            pltpu.VMEM((_NSEG * _EMBED,), jnp.float32),
            pltpu.VMEM((_CHUNK, _EMBED), jnp.float32),
            pltpu.VMEM((_CHUNK, _EMBED), jnp.float32),
            pltpu.VMEM((_CHUNK, _EMBED), jnp.float32),
            pltpu.VMEM((_CHUNK, _EMBED), jnp.float32),
            pltpu.VMEM_SHARED((_MAX_LEN, _EMBED), jnp.float32),
            pltpu.SemaphoreType.DMA,
            pltpu.SemaphoreType.DMA,
            pltpu.SemaphoreType.DMA,
            pltpu.SemaphoreType.DMA,
            pltpu.SemaphoreType.DMA,
            pltpu.SemaphoreType.DMA,
        ],
    )(sequence, segment_label, token_table, segflat, pe)
    return out.reshape(b, s, _EMBED)


# speedup vs baseline: 1.5750x; 1.5750x over previous
import functools
import math

import jax
import jax.numpy as jnp
import numpy as np
from jax import lax
from jax.experimental import pallas as pl
from jax.experimental.pallas import tpu as pltpu
from jax.experimental.pallas import tpu_sc as plsc

_VOCAB = 30522
_EMBED = 768
_MAX_LEN = 512
_NSEG = 3

_NW = 32
_CHUNK = 16
_NBUF = 3
_LANES = 16
_NVEC = _EMBED // _LANES


def _make_pe_np(d_model, max_len):
    pe = np.zeros((max_len, d_model), dtype=np.float32)
    position = np.arange(0, max_len, dtype=np.float32)[:, None]
    div_term = np.exp(
        np.arange(0, d_model, 2, dtype=np.float32) * -(math.log(10000.0) / d_model)
    )
    pe[:, 0::2] = np.sin(position * div_term)
    pe[:, 1::2] = np.cos(position * div_term)
    return pe


_PE = _make_pe_np(_EMBED, _MAX_LEN)


def _addend_body(pe_ref, seg_ref, out_ref):
    g = pl.program_id(0)
    out_ref[...] = (pe_ref[...] + seg_ref[g, :][None, :])[None]


def _build_addend(seg_table):
    pe = jnp.asarray(_PE)
    return pl.pallas_call(
        _addend_body,
        grid=(_NSEG,),
        in_specs=[
            pl.BlockSpec((_MAX_LEN, _EMBED), lambda g: (0, 0)),
            pl.BlockSpec((_NSEG, _EMBED), lambda g: (0, 0)),
        ],
        out_specs=pl.BlockSpec((1, _MAX_LEN, _EMBED), lambda g: (g, 0, 0)),
        out_shape=jax.ShapeDtypeStruct((_NSEG, _MAX_LEN, _EMBED), jnp.float32),
    )(pe, seg_table).reshape(_NSEG * _MAX_LEN, _EMBED)


def _sc_body(tok_hbm, add_hbm, tokidx_hbm, addidx_hbm, out_hbm,
             tokidx_v, addidx_v,
             tokbuf0, tokbuf1, tokbuf2, addbuf0, addbuf1, addbuf2,
             sem_t0, sem_t1, sem_t2, sem_a0, sem_a1, sem_a2,
             sem_o0, sem_o1, sem_o2,
             rows_per_w=None):
    wid = lax.axis_index("s") * 2 + lax.axis_index("c")
    base = wid * rows_per_w
    pltpu.sync_copy(tokidx_hbm.at[pl.ds(base, rows_per_w)], tokidx_v)
    pltpu.sync_copy(addidx_hbm.at[pl.ds(base, rows_per_w)], addidx_v)

    nchunk = rows_per_w // _CHUNK
    tok = [tokbuf0, tokbuf1, tokbuf2]
    add = [addbuf0, addbuf1, addbuf2]
    sem_t = [sem_t0, sem_t1, sem_t2]
    sem_a = [sem_a0, sem_a1, sem_a2]
    sem_o = [sem_o0, sem_o1, sem_o2]

    def start_gather(c):
        slot = c % _NBUF
        r0 = c * _CHUNK
        cp_t = pltpu.async_copy(
            tok_hbm.at[tokidx_v.at[pl.ds(r0, _CHUNK)]], tok[slot], sem_t[slot])
        cp_a = pltpu.async_copy(
            add_hbm.at[addidx_v.at[pl.ds(r0, _CHUNK)]], add[slot], sem_a[slot])
        return cp_t, cp_a

    in_cp = {0: start_gather(0), 1: start_gather(1)}
    out_cp = [None] * _NBUF

    for c in range(nchunk):
        slot = c % _NBUF
        cp_t, cp_a = in_cp.pop(c)
        cp_t.wait()
        cp_a.wait()

        tbuf = tok[slot]
        abuf = add[slot]

        def row_body(r, carry, tbuf=tbuf, abuf=abuf):
            for j in range(_NVEC):
                sl = pl.ds(j * _LANES, _LANES)
                plsc.addupdate(tbuf.at[r, sl], abuf[r, sl])
            return carry

        lax.fori_loop(0, _CHUNK, row_body, 0)
        out_cp[slot] = pltpu.async_copy(
            tbuf, out_hbm.at[pl.ds(base + c * _CHUNK, _CHUNK)], sem_o[slot])

        if c + 2 < nchunk:
            nslot = (c + 2) % _NBUF
            if out_cp[nslot] is not None:
                out_cp[nslot].wait()
                out_cp[nslot] = None
            in_cp[c + 2] = start_gather(c + 2)

    for cp in out_cp:
        if cp is not None:
            cp.wait()


def _sc_gather_add(token_table, addend, tok_idx, add_idx):
    n = tok_idx.shape[0]
    rows_per_w = n // _NW
    mesh = plsc.VectorSubcoreMesh(core_axis_name="c", subcore_axis_name="s")
    return pl.kernel(
        functools.partial(_sc_body, rows_per_w=rows_per_w),
        out_type=jax.ShapeDtypeStruct((n, _EMBED), jnp.float32),
        mesh=mesh,
        scratch_types=(
            [pltpu.VMEM((rows_per_w,), jnp.int32)] * 2
            + [pltpu.VMEM((_CHUNK, _EMBED), jnp.float32)] * 6
            + [pltpu.SemaphoreType.DMA] * 9
        ),
    )(token_table, addend, tok_idx, add_idx)


def kernel(sequence, segment_label, token_table, segment_table):
    b, s = sequence.shape
    addend = _build_addend(segment_table)
    tok_idx = sequence.reshape(-1)
    pos = jnp.arange(s, dtype=jnp.int32)
    add_idx = (segment_label * _MAX_LEN + pos[None, :]).reshape(-1)
    out = _sc_gather_add(token_table, addend, tok_idx, add_idx)
    return out.reshape(b, s, _EMBED)


# speedup vs baseline: 1.6407x; 1.0417x over previous
import functools
import math

import jax
import jax.numpy as jnp
import numpy as np
from jax import lax
from jax.experimental import pallas as pl
from jax.experimental.pallas import tpu as pltpu
from jax.experimental.pallas import tpu_sc as plsc

_VOCAB = 30522
_EMBED = 768
_MAX_LEN = 512
_NSEG = 3

_NW = 32
_CHUNK = 32
_LANES = 16


def _make_pe_np(d_model, max_len):
    pe = np.zeros((max_len, d_model), dtype=np.float32)
    position = np.arange(0, max_len, dtype=np.float32)[:, None]
    div_term = np.exp(
        np.arange(0, d_model, 2, dtype=np.float32) * -(math.log(10000.0) / d_model)
    )
    pe[:, 0::2] = np.sin(position * div_term)
    pe[:, 1::2] = np.cos(position * div_term)
    return pe


_PE = _make_pe_np(_EMBED, _MAX_LEN)


def _addend_body(pe_ref, seg_ref, out_ref):
    g = pl.program_id(0)
    out_ref[...] = (pe_ref[...] + seg_ref[g, :][None, :])[None]


def _build_addend(seg_table):
    pe = jnp.asarray(_PE)
    return pl.pallas_call(
        _addend_body,
        grid=(_NSEG,),
        in_specs=[
            pl.BlockSpec((_MAX_LEN, _EMBED), lambda g: (0, 0)),
            pl.BlockSpec((_NSEG, _EMBED), lambda g: (0, 0)),
        ],
        out_specs=pl.BlockSpec((1, _MAX_LEN, _EMBED), lambda g: (g, 0, 0)),
        out_shape=jax.ShapeDtypeStruct((_NSEG, _MAX_LEN, _EMBED), jnp.float32),
    )(pe, seg_table).reshape(_NSEG * _MAX_LEN, _EMBED)


def _sc_body(tok_hbm, add_hbm, tokidx_hbm, addidx_hbm, out_hbm,
             tokidx_v, addidx_v, tokbuf0, tokbuf1, addbuf0, addbuf1,
             sem_t0, sem_t1, sem_a0, sem_a1, sem_o0, sem_o1,
             rows_per_w=None):
    wid = lax.axis_index("s") * 2 + lax.axis_index("c")
    base = wid * rows_per_w
    pltpu.sync_copy(tokidx_hbm.at[pl.ds(base, rows_per_w)], tokidx_v)
    pltpu.sync_copy(addidx_hbm.at[pl.ds(base, rows_per_w)], addidx_v)

    nchunk = rows_per_w // _CHUNK
    nvec = _EMBED // _LANES
    tok = [tokbuf0, tokbuf1]
    add = [addbuf0, addbuf1]
    sem_t = [sem_t0, sem_t1]
    sem_a = [sem_a0, sem_a1]
    sem_o = [sem_o0, sem_o1]

    def start_gather(c, buf_slot):
        r0 = c * _CHUNK
        cp_t = pltpu.async_copy(
            tok_hbm.at[tokidx_v.at[pl.ds(r0, _CHUNK)]], tok[buf_slot],
            sem_t[buf_slot])
        cp_a = pltpu.async_copy(
            add_hbm.at[addidx_v.at[pl.ds(r0, _CHUNK)]], add[buf_slot],
            sem_a[buf_slot])
        return cp_t, cp_a

    in_cp = {0: start_gather(0, 0)}
    out_cp = [None, None]

    for c in range(nchunk):
        cur = c & 1
        nxt = cur ^ 1
        if c + 1 < nchunk:
            if out_cp[nxt] is not None:
                out_cp[nxt].wait()
                out_cp[nxt] = None
            in_cp[c + 1] = start_gather(c + 1, nxt)
        cp_t, cp_a = in_cp.pop(c)
        cp_t.wait()
        cp_a.wait()

        tbuf = tok[cur]
        abuf = add[cur]

        def row_body(r, carry, tbuf=tbuf, abuf=abuf):
            for j in range(nvec):
                sl = pl.ds(j * _LANES, _LANES)
                tbuf[r, sl] = tbuf[r, sl] + abuf[r, sl]
            return carry

        lax.fori_loop(0, _CHUNK, row_body, 0)
        out_cp[cur] = pltpu.async_copy(
            tbuf, out_hbm.at[pl.ds(base + c * _CHUNK, _CHUNK)], sem_o[cur])

    for cp in out_cp:
        if cp is not None:
            cp.wait()


def _sc_gather_add(token_table, addend, tok_idx, add_idx):
    n = tok_idx.shape[0]
    rows_per_w = n // _NW
    mesh = plsc.VectorSubcoreMesh(core_axis_name="c", subcore_axis_name="s")
    return pl.kernel(
        functools.partial(_sc_body, rows_per_w=rows_per_w),
        out_type=jax.ShapeDtypeStruct((n, _EMBED), jnp.float32),
        mesh=mesh,
        scratch_types=(
            [pltpu.VMEM((rows_per_w,), jnp.int32)] * 2
            + [pltpu.VMEM((_CHUNK, _EMBED), jnp.float32)] * 4
            + [pltpu.SemaphoreType.DMA] * 6
        ),
    )(token_table, addend, tok_idx, add_idx)


def kernel(sequence, segment_label, token_table, segment_table):
    b, s = sequence.shape
    addend = _build_addend(segment_table)
    tok_idx = sequence.reshape(-1)
    pos = jnp.arange(s, dtype=jnp.int32)
    add_idx = (segment_label * _MAX_LEN + pos[None, :]).reshape(-1)
    out = _sc_gather_add(token_table, addend, tok_idx, add_idx)
    return out.reshape(b, s, _EMBED)
